# baseline (speedup 1.0000x reference)
import functools
import math

import jax
import jax.numpy as jnp
import numpy as np
from jax import lax
from jax.experimental import pallas as pl
from jax.experimental.pallas import tpu as pltpu
from jax.experimental.pallas import tpu_sc as plsc

N_NODES = 10000
N_EDGES = 320000
IN_DIM = 128
HIDDEN = 256
TDIM = 64
N_GRAPHS = 16

NC = 2
NS = 16
HALF = HIDDEN // 2

N_PAD = 10240
NPT = N_PAD // NS
EPT = N_EDGES // NS
CH = 160
ROWS = 2000
GRID = N_NODES // ROWS

_mesh = plsc.VectorSubcoreMesh(core_axis_name="c", subcore_axis_name="s")


@functools.partial(
    pl.kernel,
    out_type=jax.ShapeDtypeStruct((NC, N_PAD, HALF), jnp.float32),
    mesh=_mesh,
    scratch_types=[
        pltpu.VMEM((CH,), jnp.int32),
        pltpu.VMEM((CH,), jnp.int32),
        pltpu.VMEM((CH,), jnp.int32),
        pltpu.VMEM((CH,), jnp.int32),
        pltpu.VMEM((CH, HALF), jnp.float32),
        pltpu.VMEM((CH, HALF), jnp.float32),
        pltpu.VMEM((16,), jnp.int32),
        pltpu.VMEM_SHARED((N_PAD, HALF), jnp.float32),
        pltpu.SemaphoreType.DMA,
        pltpu.SemaphoreType.DMA,
    ],
)
def _conv_kernel(y_hbm, src_hbm, dst_hbm, rng_hbm, out_hbm, src_v0, dst_v0,
                 src_v1, dst_v1, rows_v0, rows_v1, rng_v, acc_sh, sem0, sem1):
    c = lax.axis_index("c")
    s = lax.axis_index("s")
    w = c * NS + s
    nbase = s * NPT
    pltpu.sync_copy(y_hbm.at[c, pl.ds(nbase, NPT)], acc_sh.at[pl.ds(nbase, NPT)])
    pltpu.sync_copy(rng_hbm.at[pl.ds(w * 16, 16)], rng_v)
    plsc.subcore_barrier()
    rng = rng_v[...]
    gbase = rng[0]
    n_it = rng[1]

    def load_idx(i, sv, dv):
        e0 = (gbase + i) * CH
        pltpu.sync_copy(src_hbm.at[pl.ds(e0, CH)], sv)
        pltpu.sync_copy(dst_hbm.at[pl.ds(e0, CH)], dv)

    load_idx(0, src_v0, dst_v0)
    pltpu.async_copy(y_hbm.at[c].at[src_v0], rows_v0, sem0)

    def body(i, carry):
        @pl.when(i % 2 == 0)
        def _():
            load_idx(i + 1, src_v1, dst_v1)
            pltpu.async_copy(y_hbm.at[c].at[src_v1], rows_v1, sem1)
            pltpu.make_async_copy(y_hbm.at[c].at[src_v0], rows_v0, sem0).wait()
            pltpu.sync_copy(rows_v0, acc_sh.at[dst_v0], add=True)

        @pl.when(i % 2 == 1)
        def _():
            load_idx(i + 1, src_v0, dst_v0)
            pltpu.async_copy(y_hbm.at[c].at[src_v0], rows_v0, sem0)
            pltpu.make_async_copy(y_hbm.at[c].at[src_v1], rows_v1, sem1).wait()
            pltpu.sync_copy(rows_v1, acc_sh.at[dst_v1], add=True)

        return carry

    lax.fori_loop(0, n_it - 1, body, 0)

    @pl.when((n_it - 1) % 2 == 0)
    def _():
        pltpu.make_async_copy(y_hbm.at[c].at[src_v0], rows_v0, sem0).wait()
        pltpu.sync_copy(rows_v0, acc_sh.at[dst_v0], add=True)

    @pl.when((n_it - 1) % 2 == 1)
    def _():
        pltpu.make_async_copy(y_hbm.at[c].at[src_v1], rows_v1, sem1).wait()
        pltpu.sync_copy(rows_v1, acc_sh.at[dst_v1], add=True)

    plsc.subcore_barrier()
    pltpu.sync_copy(acc_sh.at[pl.ds(nbase, NPT)], out_hbm.at[c, pl.ds(nbase, NPT)])


def _t1a_body(x_ref, w1_ref, xw_ref):
    xw = jnp.dot(x_ref[...], w1_ref[...], preferred_element_type=jnp.float32)
    xw_ref[0] = xw[:, :HALF]
    xw_ref[1] = xw[:, HALF:]


_t1a = pl.pallas_call(
    _t1a_body,
    grid=(GRID,),
    in_specs=[
        pl.BlockSpec((ROWS, IN_DIM), lambda i: (i, 0)),
        pl.BlockSpec((IN_DIM, HIDDEN), lambda i: (0, 0)),
    ],
    out_specs=pl.BlockSpec((NC, ROWS, HALF), lambda i: (0, i, 0)),
    out_shape=jax.ShapeDtypeStruct((NC, N_PAD, HALF), jnp.float32),
)


def _t1b_body(xw_ref, degp_ref, y_ref, dinv_ref):
    dinv = lax.rsqrt(degp_ref[0, :, 0:1] + degp_ref[1, :, 0:1] - 1.0)
    y_ref[0] = xw_ref[0] * dinv
    y_ref[1] = xw_ref[1] * dinv
    dinv_ref[...] = dinv


_t1b = pl.pallas_call(
    _t1b_body,
    grid=(GRID,),
    in_specs=[
        pl.BlockSpec((NC, ROWS, HALF), lambda i: (0, i, 0)),
        pl.BlockSpec((NC, ROWS, HALF), lambda i: (0, i, 0)),
    ],
    out_specs=[
        pl.BlockSpec((NC, ROWS, HALF), lambda i: (0, i, 0)),
        pl.BlockSpec((ROWS, 1), lambda i: (i, 0)),
    ],
    out_shape=[
        jax.ShapeDtypeStruct((NC, N_PAD, HALF), jnp.float32),
        jax.ShapeDtypeStruct((N_NODES, 1), jnp.float32),
    ],
)


def _t2_body(acc_ref, dinv_ref, b1_ref, w2_ref, y2_ref):
    acc = jnp.concatenate([acc_ref[0], acc_ref[1]], axis=1)
    dinv = dinv_ref[...]
    h = jnp.maximum(acc * dinv + b1_ref[...], 0.0)
    xw = jnp.dot(h, w2_ref[...], preferred_element_type=jnp.float32)
    y2 = xw * dinv
    y2_ref[0] = y2[:, :HALF]
    y2_ref[1] = y2[:, HALF:]


_t2 = pl.pallas_call(
    _t2_body,
    grid=(GRID,),
    in_specs=[
        pl.BlockSpec((NC, ROWS, HALF), lambda i: (0, i, 0)),
        pl.BlockSpec((ROWS, 1), lambda i: (i, 0)),
        pl.BlockSpec((1, HIDDEN), lambda i: (0, 0)),
        pl.BlockSpec((HIDDEN, HIDDEN), lambda i: (0, 0)),
    ],
    out_specs=pl.BlockSpec((NC, ROWS, HALF), lambda i: (0, i, 0)),
    out_shape=jax.ShapeDtypeStruct((NC, N_PAD, HALF), jnp.float32),
)


def _temb_body(t_ref, tw_ref, tb_ref, gemb_ref):
    half = TDIM // 2
    k = lax.broadcasted_iota(jnp.int32, (1, half), 1).astype(jnp.float32)
    freqs = jnp.exp(k * (-math.log(10000.0) / half))
    args = t_ref[...] * freqs
    temb = jnp.concatenate([jnp.sin(args), jnp.cos(args)], axis=1)
    a = jnp.dot(temb, tw_ref[...], preferred_element_type=jnp.float32) + tb_ref[...]
    gemb_ref[...] = a * (1.0 / (1.0 + jnp.exp(-a)))


_temb = pl.pallas_call(
    _temb_body,
    out_shape=jax.ShapeDtypeStruct((N_GRAPHS, HIDDEN), jnp.float32),
)


def _t3_body(acc_ref, dinv_ref, b2_ref, gemb_ref, bv_ref, mw1_ref, mb1_ref,
             mw2_ref, mb2_ref, out_ref):
    acc = jnp.concatenate([acc_ref[0], acc_ref[1]], axis=1)
    h = jnp.maximum(acc * dinv_ref[...] + b2_ref[...], 0.0)
    gid = lax.broadcasted_iota(jnp.int32, (1, N_GRAPHS), 1)
    onehot = (bv_ref[...] == gid).astype(jnp.float32)
    h = h + jnp.dot(onehot, gemb_ref[...], preferred_element_type=jnp.float32)
    a1 = jnp.dot(h, mw1_ref[...], preferred_element_type=jnp.float32) + mb1_ref[...]
    h3 = a1 * (1.0 / (1.0 + jnp.exp(-a1)))
    out_ref[...] = jnp.dot(h3, mw2_ref[...], preferred_element_type=jnp.float32) + mb2_ref[...]


_t3 = pl.pallas_call(
    _t3_body,
    grid=(GRID,),
    in_specs=[
        pl.BlockSpec((NC, ROWS, HALF), lambda i: (0, i, 0)),
        pl.BlockSpec((ROWS, 1), lambda i: (i, 0)),
        pl.BlockSpec((1, HIDDEN), lambda i: (0, 0)),
        pl.BlockSpec((N_GRAPHS, HIDDEN), lambda i: (0, 0)),
        pl.BlockSpec((ROWS, 1), lambda i: (i, 0)),
        pl.BlockSpec((HIDDEN, HIDDEN), lambda i: (0, 0)),
        pl.BlockSpec((1, HIDDEN), lambda i: (0, 0)),
        pl.BlockSpec((HIDDEN, IN_DIM), lambda i: (0, 0)),
        pl.BlockSpec((1, IN_DIM), lambda i: (0, 0)),
    ],
    out_specs=pl.BlockSpec((ROWS, IN_DIM), lambda i: (i, 0)),
    out_shape=jax.ShapeDtypeStruct((N_NODES, IN_DIM), jnp.float32),
)


def kernel(x, W1, b1, W2, b2, tW, tb, mW1, mb1, mW2, mb2, edge_index, t_graph, batch_vec):
    src = edge_index[0]
    dst = edge_index[1]

    n_chunks = N_EDGES // CH
    cpt = n_chunks // NS
    rng_conv = np.zeros((NC * NS, 16), np.int32)
    for cc in range(NC):
        for ss in range(NS):
            rng_conv[cc * NS + ss, 0] = ss * cpt
            rng_conv[cc * NS + ss, 1] = cpt
    rng_deg = np.zeros((NC * NS, 16), np.int32)
    base = 0
    for ww in range(NC * NS):
        nit = n_chunks // (NC * NS) + (1 if ww < n_chunks % (NC * NS) else 0)
        rng_deg[ww, 0] = base
        rng_deg[ww, 1] = nit
        base += nit
    rng_conv = jnp.asarray(rng_conv.reshape(-1))
    rng_deg = jnp.asarray(rng_deg.reshape(-1))

    src_seq = jnp.asarray(np.arange(N_EDGES, dtype=np.int32) % N_PAD)
    xw1 = _t1a(x, W1)
    degp = _conv_kernel(jnp.ones((NC, N_PAD, HALF), jnp.float32), src_seq,
                        dst, rng_deg)
    gemb = _temb(t_graph.astype(jnp.float32).reshape(N_GRAPHS, 1), tW,
                 tb.reshape(1, HIDDEN))
    y1, dinv = _t1b(xw1, degp)
    acc1 = _conv_kernel(y1, src, dst, rng_conv)
    y2 = _t2(acc1, dinv, b1.reshape(1, HIDDEN), W2)
    acc2 = _conv_kernel(y2, src, dst, rng_conv)
    return _t3(acc2, dinv, b2.reshape(1, HIDDEN), gemb,
               batch_vec.reshape(N_NODES, 1), mW1, mb1.reshape(1, HIDDEN),
               mW2, mb2.reshape(1, IN_DIM))


# speedup vs baseline: 1.0075x; 1.0075x over previous
import functools
import math

import jax
import jax.numpy as jnp
import numpy as np
from jax import lax
from jax.experimental import pallas as pl
from jax.experimental.pallas import tpu as pltpu
from jax.experimental.pallas import tpu_sc as plsc

N_NODES = 10000
N_EDGES = 320000
IN_DIM = 128
HIDDEN = 256
TDIM = 64
N_GRAPHS = 16

NC = 2
NS = 16
HALF = HIDDEN // 2

N_PAD = 10240
NPT = N_PAD // NS
EPT = N_EDGES // NS
CH = 160
ROWS = 5000
GRID = N_NODES // ROWS

_mesh = plsc.VectorSubcoreMesh(core_axis_name="c", subcore_axis_name="s")


@functools.partial(
    pl.kernel,
    out_type=jax.ShapeDtypeStruct((NC, N_PAD, HALF), jnp.float32),
    mesh=_mesh,
    scratch_types=[
        pltpu.VMEM((CH,), jnp.int32),
        pltpu.VMEM((CH,), jnp.int32),
        pltpu.VMEM((CH,), jnp.int32),
        pltpu.VMEM((CH,), jnp.int32),
        pltpu.VMEM((CH, HALF), jnp.float32),
        pltpu.VMEM((CH, HALF), jnp.float32),
        pltpu.VMEM((16,), jnp.int32),
        pltpu.VMEM_SHARED((N_PAD, HALF), jnp.float32),
        pltpu.SemaphoreType.DMA,
        pltpu.SemaphoreType.DMA,
    ],
)
def _conv_kernel(y_hbm, src_hbm, dst_hbm, rng_hbm, out_hbm, src_v0, dst_v0,
                 src_v1, dst_v1, rows_v0, rows_v1, rng_v, acc_sh, sem0, sem1):
    c = lax.axis_index("c")
    s = lax.axis_index("s")
    w = c * NS + s
    nbase = s * NPT
    pltpu.sync_copy(y_hbm.at[c, pl.ds(nbase, NPT)], acc_sh.at[pl.ds(nbase, NPT)])
    pltpu.sync_copy(rng_hbm.at[pl.ds(w * 16, 16)], rng_v)
    plsc.subcore_barrier()
    rng = rng_v[...]
    gbase = rng[0]
    n_it = rng[1]

    def load_idx(i, sv, dv):
        e0 = (gbase + i) * CH
        pltpu.sync_copy(src_hbm.at[pl.ds(e0, CH)], sv)
        pltpu.sync_copy(dst_hbm.at[pl.ds(e0, CH)], dv)

    load_idx(0, src_v0, dst_v0)
    pltpu.async_copy(y_hbm.at[c].at[src_v0], rows_v0, sem0)

    def body(i, carry):
        @pl.when(i % 2 == 0)
        def _():
            load_idx(i + 1, src_v1, dst_v1)
            pltpu.async_copy(y_hbm.at[c].at[src_v1], rows_v1, sem1)
            pltpu.make_async_copy(y_hbm.at[c].at[src_v0], rows_v0, sem0).wait()
            pltpu.sync_copy(rows_v0, acc_sh.at[dst_v0], add=True)

        @pl.when(i % 2 == 1)
        def _():
            load_idx(i + 1, src_v0, dst_v0)
            pltpu.async_copy(y_hbm.at[c].at[src_v0], rows_v0, sem0)
            pltpu.make_async_copy(y_hbm.at[c].at[src_v1], rows_v1, sem1).wait()
            pltpu.sync_copy(rows_v1, acc_sh.at[dst_v1], add=True)

        return carry

    lax.fori_loop(0, n_it - 1, body, 0)

    @pl.when((n_it - 1) % 2 == 0)
    def _():
        pltpu.make_async_copy(y_hbm.at[c].at[src_v0], rows_v0, sem0).wait()
        pltpu.sync_copy(rows_v0, acc_sh.at[dst_v0], add=True)

    @pl.when((n_it - 1) % 2 == 1)
    def _():
        pltpu.make_async_copy(y_hbm.at[c].at[src_v1], rows_v1, sem1).wait()
        pltpu.sync_copy(rows_v1, acc_sh.at[dst_v1], add=True)

    plsc.subcore_barrier()
    pltpu.sync_copy(acc_sh.at[pl.ds(nbase, NPT)], out_hbm.at[c, pl.ds(nbase, NPT)])


def _t1a_body(x_ref, w1_ref, xw_ref):
    xw = jnp.dot(x_ref[...], w1_ref[...], preferred_element_type=jnp.float32)
    xw_ref[0] = xw[:, :HALF]
    xw_ref[1] = xw[:, HALF:]


_t1a = pl.pallas_call(
    _t1a_body,
    grid=(GRID,),
    in_specs=[
        pl.BlockSpec((ROWS, IN_DIM), lambda i: (i, 0)),
        pl.BlockSpec((IN_DIM, HIDDEN), lambda i: (0, 0)),
    ],
    out_specs=pl.BlockSpec((NC, ROWS, HALF), lambda i: (0, i, 0)),
    out_shape=jax.ShapeDtypeStruct((NC, N_PAD, HALF), jnp.float32),
)


def _t1b_body(xw_ref, degp_ref, y_ref, dinv_ref):
    dinv = lax.rsqrt(degp_ref[0, :, 0:1] + degp_ref[1, :, 0:1] - 1.0)
    y_ref[0] = xw_ref[0] * dinv
    y_ref[1] = xw_ref[1] * dinv
    dinv_ref[...] = dinv


_t1b = pl.pallas_call(
    _t1b_body,
    grid=(GRID,),
    in_specs=[
        pl.BlockSpec((NC, ROWS, HALF), lambda i: (0, i, 0)),
        pl.BlockSpec((NC, ROWS, HALF), lambda i: (0, i, 0)),
    ],
    out_specs=[
        pl.BlockSpec((NC, ROWS, HALF), lambda i: (0, i, 0)),
        pl.BlockSpec((ROWS, 1), lambda i: (i, 0)),
    ],
    out_shape=[
        jax.ShapeDtypeStruct((NC, N_PAD, HALF), jnp.float32),
        jax.ShapeDtypeStruct((N_NODES, 1), jnp.float32),
    ],
)


def _t2_body(acc_ref, dinv_ref, b1_ref, w2_ref, y2_ref):
    acc = jnp.concatenate([acc_ref[0], acc_ref[1]], axis=1)
    dinv = dinv_ref[...]
    h = jnp.maximum(acc * dinv + b1_ref[...], 0.0)
    xw = jnp.dot(h, w2_ref[...], preferred_element_type=jnp.float32)
    y2 = xw * dinv
    y2_ref[0] = y2[:, :HALF]
    y2_ref[1] = y2[:, HALF:]


_t2 = pl.pallas_call(
    _t2_body,
    grid=(GRID,),
    in_specs=[
        pl.BlockSpec((NC, ROWS, HALF), lambda i: (0, i, 0)),
        pl.BlockSpec((ROWS, 1), lambda i: (i, 0)),
        pl.BlockSpec((1, HIDDEN), lambda i: (0, 0)),
        pl.BlockSpec((HIDDEN, HIDDEN), lambda i: (0, 0)),
    ],
    out_specs=pl.BlockSpec((NC, ROWS, HALF), lambda i: (0, i, 0)),
    out_shape=jax.ShapeDtypeStruct((NC, N_PAD, HALF), jnp.float32),
)


def _temb_body(t_ref, tw_ref, tb_ref, gemb_ref):
    half = TDIM // 2
    k = lax.broadcasted_iota(jnp.int32, (1, half), 1).astype(jnp.float32)
    freqs = jnp.exp(k * (-math.log(10000.0) / half))
    args = t_ref[...] * freqs
    temb = jnp.concatenate([jnp.sin(args), jnp.cos(args)], axis=1)
    a = jnp.dot(temb, tw_ref[...], preferred_element_type=jnp.float32) + tb_ref[...]
    gemb_ref[...] = a * (1.0 / (1.0 + jnp.exp(-a)))


_temb = pl.pallas_call(
    _temb_body,
    out_shape=jax.ShapeDtypeStruct((N_GRAPHS, HIDDEN), jnp.float32),
)


def _t3_body(acc_ref, dinv_ref, b2_ref, gemb_ref, bv_ref, mw1_ref, mb1_ref,
             mw2_ref, mb2_ref, out_ref):
    acc = jnp.concatenate([acc_ref[0], acc_ref[1]], axis=1)
    h = jnp.maximum(acc * dinv_ref[...] + b2_ref[...], 0.0)
    gid = lax.broadcasted_iota(jnp.int32, (1, N_GRAPHS), 1)
    onehot = (bv_ref[...] == gid).astype(jnp.float32)
    h = h + jnp.dot(onehot, gemb_ref[...], preferred_element_type=jnp.float32)
    a1 = jnp.dot(h, mw1_ref[...], preferred_element_type=jnp.float32) + mb1_ref[...]
    h3 = a1 * (1.0 / (1.0 + jnp.exp(-a1)))
    out_ref[...] = jnp.dot(h3, mw2_ref[...], preferred_element_type=jnp.float32) + mb2_ref[...]


_t3 = pl.pallas_call(
    _t3_body,
    grid=(GRID,),
    in_specs=[
        pl.BlockSpec((NC, ROWS, HALF), lambda i: (0, i, 0)),
        pl.BlockSpec((ROWS, 1), lambda i: (i, 0)),
        pl.BlockSpec((1, HIDDEN), lambda i: (0, 0)),
        pl.BlockSpec((N_GRAPHS, HIDDEN), lambda i: (0, 0)),
        pl.BlockSpec((ROWS, 1), lambda i: (i, 0)),
        pl.BlockSpec((HIDDEN, HIDDEN), lambda i: (0, 0)),
        pl.BlockSpec((1, HIDDEN), lambda i: (0, 0)),
        pl.BlockSpec((HIDDEN, IN_DIM), lambda i: (0, 0)),
        pl.BlockSpec((1, IN_DIM), lambda i: (0, 0)),
    ],
    out_specs=pl.BlockSpec((ROWS, IN_DIM), lambda i: (i, 0)),
    out_shape=jax.ShapeDtypeStruct((N_NODES, IN_DIM), jnp.float32),
)


def kernel(x, W1, b1, W2, b2, tW, tb, mW1, mb1, mW2, mb2, edge_index, t_graph, batch_vec):
    src = edge_index[0]
    dst = edge_index[1]

    n_chunks = N_EDGES // CH
    cpt = n_chunks // NS
    rng_conv = np.zeros((NC * NS, 16), np.int32)
    for cc in range(NC):
        for ss in range(NS):
            rng_conv[cc * NS + ss, 0] = ss * cpt
            rng_conv[cc * NS + ss, 1] = cpt
    rng_deg = np.zeros((NC * NS, 16), np.int32)
    base = 0
    for ww in range(NC * NS):
        nit = n_chunks // (NC * NS) + (1 if ww < n_chunks % (NC * NS) else 0)
        rng_deg[ww, 0] = base
        rng_deg[ww, 1] = nit
        base += nit
    rng_conv = jnp.asarray(rng_conv.reshape(-1))
    rng_deg = jnp.asarray(rng_deg.reshape(-1))

    src_seq = jnp.asarray(np.arange(N_EDGES, dtype=np.int32) % N_PAD)
    degp = _conv_kernel(jnp.ones((NC, N_PAD, HALF), jnp.float32), src_seq,
                        dst, rng_deg)
    gemb = _temb(t_graph.astype(jnp.float32).reshape(N_GRAPHS, 1), tW,
                 tb.reshape(1, HIDDEN))
    y1, dinv = _t1b(_t1a(x, W1), degp)
    acc1 = _conv_kernel(y1, src, dst, rng_conv)
    y2 = _t2(acc1, dinv, b1.reshape(1, HIDDEN), W2)
    acc2 = _conv_kernel(y2, src, dst, rng_conv)
    return _t3(acc2, dinv, b2.reshape(1, HIDDEN), gemb,
               batch_vec.reshape(N_NODES, 1), mW1, mb1.reshape(1, HIDDEN),
               mW2, mb2.reshape(1, IN_DIM))


# speedup vs baseline: 1.0080x; 1.0005x over previous
import functools
import math

import jax
import jax.numpy as jnp
import numpy as np
from jax import lax
from jax.experimental import pallas as pl
from jax.experimental.pallas import tpu as pltpu
from jax.experimental.pallas import tpu_sc as plsc

N_NODES = 10000
N_EDGES = 320000
IN_DIM = 128
HIDDEN = 256
TDIM = 64
N_GRAPHS = 16

NC = 2
NS = 16
HALF = HIDDEN // 2

N_PAD = 10240
NPT = N_PAD // NS
EPT = N_EDGES // NS
CH = 160
ROWS = 5000
GRID = N_NODES // ROWS

_mesh = plsc.VectorSubcoreMesh(core_axis_name="c", subcore_axis_name="s")


@functools.partial(
    pl.kernel,
    out_type=jax.ShapeDtypeStruct((NC, N_PAD, HALF), jnp.float32),
    mesh=_mesh,
    scratch_types=[
        pltpu.VMEM((CH,), jnp.int32),
        pltpu.VMEM((CH,), jnp.int32),
        pltpu.VMEM((CH,), jnp.int32),
        pltpu.VMEM((CH,), jnp.int32),
        pltpu.VMEM((CH, HALF), jnp.float32),
        pltpu.VMEM((CH, HALF), jnp.float32),
        pltpu.VMEM((16,), jnp.int32),
        pltpu.VMEM_SHARED((N_PAD, HALF), jnp.float32),
        pltpu.SemaphoreType.DMA,
        pltpu.SemaphoreType.DMA,
    ],
)
def _conv_kernel(y_hbm, src_hbm, dst_hbm, rng_hbm, out_hbm, src_v0, dst_v0,
                 src_v1, dst_v1, rows_v0, rows_v1, rng_v, acc_sh, sem0, sem1):
    c = lax.axis_index("c")
    s = lax.axis_index("s")
    w = c * NS + s
    nbase = s * NPT
    pltpu.sync_copy(y_hbm.at[c, pl.ds(nbase, NPT)], acc_sh.at[pl.ds(nbase, NPT)])
    pltpu.sync_copy(rng_hbm.at[pl.ds(w * 16, 16)], rng_v)
    plsc.subcore_barrier()
    rng = rng_v[...]
    gbase = rng[0]
    n_it = rng[1]

    def load_idx(i, sv, dv):
        e0 = (gbase + i) * CH
        pltpu.sync_copy(src_hbm.at[pl.ds(e0, CH)], sv)
        pltpu.sync_copy(dst_hbm.at[pl.ds(e0, CH)], dv)

    load_idx(0, src_v0, dst_v0)
    pltpu.async_copy(y_hbm.at[c].at[src_v0], rows_v0, sem0)

    def body(i, carry):
        @pl.when(i % 2 == 0)
        def _():
            load_idx(i + 1, src_v1, dst_v1)
            pltpu.async_copy(y_hbm.at[c].at[src_v1], rows_v1, sem1)
            pltpu.make_async_copy(y_hbm.at[c].at[src_v0], rows_v0, sem0).wait()
            pltpu.sync_copy(rows_v0, acc_sh.at[dst_v0], add=True)

        @pl.when(i % 2 == 1)
        def _():
            load_idx(i + 1, src_v0, dst_v0)
            pltpu.async_copy(y_hbm.at[c].at[src_v0], rows_v0, sem0)
            pltpu.make_async_copy(y_hbm.at[c].at[src_v1], rows_v1, sem1).wait()
            pltpu.sync_copy(rows_v1, acc_sh.at[dst_v1], add=True)

        return carry

    lax.fori_loop(0, n_it - 1, body, 0)

    @pl.when((n_it - 1) % 2 == 0)
    def _():
        pltpu.make_async_copy(y_hbm.at[c].at[src_v0], rows_v0, sem0).wait()
        pltpu.sync_copy(rows_v0, acc_sh.at[dst_v0], add=True)

    @pl.when((n_it - 1) % 2 == 1)
    def _():
        pltpu.make_async_copy(y_hbm.at[c].at[src_v1], rows_v1, sem1).wait()
        pltpu.sync_copy(rows_v1, acc_sh.at[dst_v1], add=True)

    plsc.subcore_barrier()
    pltpu.sync_copy(acc_sh.at[pl.ds(nbase, NPT)], out_hbm.at[c, pl.ds(nbase, NPT)])


def _t1a_body(x_ref, w1_ref, xw_ref):
    xw = jnp.dot(x_ref[...], w1_ref[...], preferred_element_type=jnp.float32)
    xw_ref[0] = xw[:, :HALF]
    xw_ref[1] = xw[:, HALF:]


_t1a = pl.pallas_call(
    _t1a_body,
    grid=(GRID,),
    in_specs=[
        pl.BlockSpec((ROWS, IN_DIM), lambda i: (i, 0)),
        pl.BlockSpec((IN_DIM, HIDDEN), lambda i: (0, 0)),
    ],
    out_specs=pl.BlockSpec((NC, ROWS, HALF), lambda i: (0, i, 0)),
    out_shape=jax.ShapeDtypeStruct((NC, N_PAD, HALF), jnp.float32),
)


def _t1b_body(xw_ref, degp_ref, y_ref, dinv_ref):
    dinv = lax.rsqrt(degp_ref[0, :, 0:1] + degp_ref[1, :, 0:1] - 1.0)
    y_ref[0] = xw_ref[0] * dinv
    y_ref[1] = xw_ref[1] * dinv
    dinv_ref[...] = dinv


_t1b = pl.pallas_call(
    _t1b_body,
    grid=(GRID,),
    in_specs=[
        pl.BlockSpec((NC, ROWS, HALF), lambda i: (0, i, 0)),
        pl.BlockSpec((NC, ROWS, HALF), lambda i: (0, i, 0)),
    ],
    out_specs=[
        pl.BlockSpec((NC, ROWS, HALF), lambda i: (0, i, 0)),
        pl.BlockSpec((ROWS, 1), lambda i: (i, 0)),
    ],
    out_shape=[
        jax.ShapeDtypeStruct((NC, N_PAD, HALF), jnp.float32),
        jax.ShapeDtypeStruct((N_NODES, 1), jnp.float32),
    ],
)


def _t2_body(acc_ref, dinv_ref, b1_ref, w2_ref, y2_ref):
    acc = jnp.concatenate([acc_ref[0], acc_ref[1]], axis=1)
    dinv = dinv_ref[...]
    h = jnp.maximum(acc * dinv + b1_ref[...], 0.0)
    xw = jnp.dot(h, w2_ref[...], preferred_element_type=jnp.float32)
    y2 = xw * dinv
    y2_ref[0] = y2[:, :HALF]
    y2_ref[1] = y2[:, HALF:]


_t2 = pl.pallas_call(
    _t2_body,
    grid=(GRID,),
    in_specs=[
        pl.BlockSpec((NC, ROWS, HALF), lambda i: (0, i, 0)),
        pl.BlockSpec((ROWS, 1), lambda i: (i, 0)),
        pl.BlockSpec((1, HIDDEN), lambda i: (0, 0)),
        pl.BlockSpec((HIDDEN, HIDDEN), lambda i: (0, 0)),
    ],
    out_specs=pl.BlockSpec((NC, ROWS, HALF), lambda i: (0, i, 0)),
    out_shape=jax.ShapeDtypeStruct((NC, N_PAD, HALF), jnp.float32),
)


def _t3_body(acc_ref, dinv_ref, b2_ref, t_ref, tw_ref, tb_ref, bv_ref,
             mw1_ref, mb1_ref, mw2_ref, mb2_ref, out_ref):
    half = TDIM // 2
    k = lax.broadcasted_iota(jnp.int32, (1, half), 1).astype(jnp.float32)
    freqs = jnp.exp(k * (-math.log(10000.0) / half))
    args = t_ref[...] * freqs
    temb = jnp.concatenate([jnp.sin(args), jnp.cos(args)], axis=1)
    a = jnp.dot(temb, tw_ref[...], preferred_element_type=jnp.float32) + tb_ref[...]
    gemb = a * (1.0 / (1.0 + jnp.exp(-a)))
    acc = jnp.concatenate([acc_ref[0], acc_ref[1]], axis=1)
    h = jnp.maximum(acc * dinv_ref[...] + b2_ref[...], 0.0)
    gid = lax.broadcasted_iota(jnp.int32, (1, N_GRAPHS), 1)
    onehot = (bv_ref[...] == gid).astype(jnp.float32)
    h = h + jnp.dot(onehot, gemb, preferred_element_type=jnp.float32)
    a1 = jnp.dot(h, mw1_ref[...], preferred_element_type=jnp.float32) + mb1_ref[...]
    h3 = a1 * (1.0 / (1.0 + jnp.exp(-a1)))
    out_ref[...] = jnp.dot(h3, mw2_ref[...], preferred_element_type=jnp.float32) + mb2_ref[...]


_t3 = pl.pallas_call(
    _t3_body,
    grid=(GRID,),
    in_specs=[
        pl.BlockSpec((NC, ROWS, HALF), lambda i: (0, i, 0)),
        pl.BlockSpec((ROWS, 1), lambda i: (i, 0)),
        pl.BlockSpec((1, HIDDEN), lambda i: (0, 0)),
        pl.BlockSpec((N_GRAPHS, 1), lambda i: (0, 0)),
        pl.BlockSpec((TDIM, HIDDEN), lambda i: (0, 0)),
        pl.BlockSpec((1, HIDDEN), lambda i: (0, 0)),
        pl.BlockSpec((ROWS, 1), lambda i: (i, 0)),
        pl.BlockSpec((HIDDEN, HIDDEN), lambda i: (0, 0)),
        pl.BlockSpec((1, HIDDEN), lambda i: (0, 0)),
        pl.BlockSpec((HIDDEN, IN_DIM), lambda i: (0, 0)),
        pl.BlockSpec((1, IN_DIM), lambda i: (0, 0)),
    ],
    out_specs=pl.BlockSpec((ROWS, IN_DIM), lambda i: (i, 0)),
    out_shape=jax.ShapeDtypeStruct((N_NODES, IN_DIM), jnp.float32),
)


def kernel(x, W1, b1, W2, b2, tW, tb, mW1, mb1, mW2, mb2, edge_index, t_graph, batch_vec):
    src = edge_index[0]
    dst = edge_index[1]

    n_chunks = N_EDGES // CH
    cpt = n_chunks // NS
    rng_conv = np.zeros((NC * NS, 16), np.int32)
    for cc in range(NC):
        for ss in range(NS):
            rng_conv[cc * NS + ss, 0] = ss * cpt
            rng_conv[cc * NS + ss, 1] = cpt
    rng_deg = np.zeros((NC * NS, 16), np.int32)
    base = 0
    for ww in range(NC * NS):
        nit = n_chunks // (NC * NS) + (1 if ww < n_chunks % (NC * NS) else 0)
        rng_deg[ww, 0] = base
        rng_deg[ww, 1] = nit
        base += nit
    rng_conv = jnp.asarray(rng_conv.reshape(-1))
    rng_deg = jnp.asarray(rng_deg.reshape(-1))

    src_seq = jnp.asarray(np.arange(N_EDGES, dtype=np.int32) % N_PAD)
    degp = _conv_kernel(jnp.ones((NC, N_PAD, HALF), jnp.float32), src_seq,
                        dst, rng_deg)
    y1, dinv = _t1b(_t1a(x, W1), degp)
    acc1 = _conv_kernel(y1, src, dst, rng_conv)
    y2 = _t2(acc1, dinv, b1.reshape(1, HIDDEN), W2)
    acc2 = _conv_kernel(y2, src, dst, rng_conv)
    return _t3(acc2, dinv, b2.reshape(1, HIDDEN),
               t_graph.astype(jnp.float32).reshape(N_GRAPHS, 1), tW,
               tb.reshape(1, HIDDEN), batch_vec.reshape(N_NODES, 1), mW1,
               mb1.reshape(1, HIDDEN), mW2, mb2.reshape(1, IN_DIM))
